# baseline (speedup 1.0000x reference)
import functools

import jax
import jax.numpy as jnp
from jax import lax
from jax.experimental import pallas as pl
from jax.experimental.pallas import tpu as pltpu
from jax.experimental.pallas import tpu_sc as plsc

_TILE_K = 2048


def _sc_gather_rows(table, idx):
    B = idx.shape[0]
    D = table.shape[1]
    info = plsc.get_sparse_core_info()
    nw = info.num_cores * info.num_subcores
    b_per_w = B // nw
    mesh = plsc.VectorSubcoreMesh(core_axis_name="c", subcore_axis_name="s")

    @functools.partial(
        pl.kernel, mesh=mesh,
        out_type=jax.ShapeDtypeStruct((B, D), jnp.float32),
        scratch_types=[
            pltpu.VMEM((b_per_w,), jnp.int32),
            pltpu.VMEM((b_per_w, D), jnp.float32),
            pltpu.SemaphoreType.DMA,
        ],
    )
    def gather_k(table_hbm, idx_hbm, out_hbm, idx_v, rows_v, sem):
        wid = lax.axis_index("s") * info.num_cores + lax.axis_index("c")
        base = wid * b_per_w
        pltpu.sync_copy(idx_hbm.at[pl.ds(base, b_per_w)], idx_v)
        pltpu.async_copy(table_hbm.at[idx_v], rows_v, sem).wait()
        pltpu.sync_copy(rows_v, out_hbm.at[pl.ds(base, b_per_w)])

    return gather_k(table, idx)


def _float_pred(x):
    xb = lax.bitcast_convert_type(x, jnp.int32)
    pb = jnp.where(xb == 0, jnp.int32(-2147483647),
                   jnp.where(xb > 0, xb - 1, xb + 1))
    return lax.bitcast_convert_type(pb, jnp.float32)


def _mrr_body(bands_ref, m1_ref, m2_ref, gath_ref, gt_ref, out_ref,
              sgt_ref, sgm_ref, cnt_ref, bufa_ref, bufb_ref, *, K, T, NT):
    k = pl.program_id(0)
    Q = m1_ref.shape[0]

    @pl.when(k == 0)
    def _groundtruth_scores():
        rows = lax.broadcasted_iota(jnp.int32, (Q, Q), 0)
        colq = lax.broadcasted_iota(jnp.int32, (Q, Q), 1)
        P = lax.dot_general(
            m1_ref[...], gath_ref[...],
            dimension_numbers=(((1,), (1,)), ((), ())),
            preferred_element_type=jnp.float32,
        )
        sg = jnp.sum(jnp.where(rows == colq, P, 0.0), axis=1, keepdims=True)
        sgt_ref[...] = sg
        sgm_ref[...] = _float_pred(sg)
        cnt_ref[...] = jnp.zeros_like(cnt_ref)
        bufb_ref[...] = jnp.full_like(bufb_ref, -jnp.inf)

    def count(dst_ref, src_ref, last):
        below = gt_ref[...] >= k * T
        thr = jnp.where(below, sgm_ref[...], sgt_ref[...])
        acc = jnp.zeros((Q, 1), jnp.float32)
        nc = 16
        C = T // nc
        for c in range(nc):
            sl = pl.ds(c * C, C)
            dst_ref[:, sl] = lax.dot_general(
                m1_ref[...], m2_ref[sl, :],
                dimension_numbers=(((1,), (1,)), ((), ())),
                preferred_element_type=jnp.float32,
            )
            sim = src_ref[:, sl]
            cmp = sim > thr
            if last:
                lane = c * C + lax.broadcasted_iota(jnp.int32, (Q, C), 1)
                cmp = jnp.logical_and(cmp, lane < K - (NT - 1) * T)
            acc = acc + jnp.sum(jnp.where(cmp, 1.0, 0.0), axis=1,
                                keepdims=True)
        cnt_ref[...] += acc
        t0 = jnp.maximum(k - 1, 0)
        lo = bands_ref[t0, 0]
        nch = jnp.where(k == 0, 0, bands_ref[t0, 1])

        def chunk(c, carry):
            r0 = pl.multiple_of(lo + c * 8, 8)
            simb = src_ref[pl.ds(r0, 8), :]
            sgb = sgt_ref[pl.ds(r0, 8), :]
            gth = gt_ref[pl.ds(r0, 8), :] - t0 * T
            gth = jnp.where(gth >= T, 0, gth)
            lane8 = lax.broadcasted_iota(jnp.int32, (8, T), 1)
            tie = jnp.logical_and(simb == sgb, lane8 < gth)
            cnt_ref[pl.ds(r0, 8), :] += jnp.sum(
                jnp.where(tie, 1.0, 0.0), axis=1, keepdims=True)
            return carry

        lax.fori_loop(0, nch, chunk, 0)

    def phase(dst_ref, src_ref):
        @pl.when(k < NT)
        def _steady():
            count(dst_ref, src_ref, last=False)

        @pl.when(k == NT)
        def _last():
            count(dst_ref, src_ref, last=True)

    @pl.when(k % 2 == 0)
    def _even():
        phase(bufa_ref, bufb_ref)

    @pl.when(k % 2 == 1)
    def _odd():
        phase(bufb_ref, bufa_ref)

    @pl.when(k == NT)
    def _finalize():
        ranks = cnt_ref[...] + 1.0
        out_ref[...] = jnp.mean(1.0 / ranks).reshape(1, 1)


def _mrr_g1(m1, m2, g):
    Q, D = m1.shape
    K = m2.shape[0]
    T = _TILE_K
    nt = pl.cdiv(K, T)

    order = jnp.argsort(g).astype(jnp.int32)
    g_sorted = jnp.take(g, order)

    gathered = _sc_gather_rows(m2, g_sorted)
    m1s = _sc_gather_rows(m1, order)

    tt = jnp.arange(nt, dtype=jnp.int32)
    lo = jnp.searchsorted(g_sorted, tt * T).astype(jnp.int32)
    hi = jnp.searchsorted(g_sorted, (tt + 1) * T).astype(jnp.int32)
    lo8 = lo // 8 * 8
    hi8 = jnp.minimum((hi + 7) // 8 * 8, Q)
    nch = jnp.maximum(hi8 - lo8, 0) // 8
    bands = jnp.stack([lo8, nch], axis=1)

    body = functools.partial(_mrr_body, K=K, T=T, NT=nt)
    grid_spec = pltpu.PrefetchScalarGridSpec(
        num_scalar_prefetch=1,
        grid=(nt + 1,),
        in_specs=[
            pl.BlockSpec((Q, D), lambda k, b: (0, 0)),
            pl.BlockSpec((T, D), lambda k, b: (jnp.minimum(k, nt - 1), 0)),
            pl.BlockSpec((Q, D), lambda k, b: (0, 0)),
            pl.BlockSpec((Q, 1), lambda k, b: (0, 0)),
        ],
        out_specs=pl.BlockSpec((1, 1), lambda k, b: (0, 0)),
        scratch_shapes=[
            pltpu.VMEM((Q, 1), jnp.float32),
            pltpu.VMEM((Q, 1), jnp.float32),
            pltpu.VMEM((Q, 1), jnp.float32),
            pltpu.VMEM((Q, T), jnp.float32),
            pltpu.VMEM((Q, T), jnp.float32),
        ],
    )
    out = pl.pallas_call(
        body,
        grid_spec=grid_spec,
        out_shape=jax.ShapeDtypeStruct((1, 1), jnp.float32),
    )(bands, m1s, m2, gathered, g_sorted.reshape(Q, 1))
    return out[0, 0]


def kernel(modality1_features, modality2_features, groundtruth_all_indices):
    gt = groundtruth_all_indices.astype(jnp.int32)
    Q, G = gt.shape
    if G != 1:
        raise NotImplementedError(
            "this problem's fixed shapes have one groundtruth per query")
    return _mrr_g1(modality1_features, modality2_features, gt[:, 0])


# speedup vs baseline: 1.3485x; 1.3485x over previous
import functools

import jax
import jax.numpy as jnp
from jax import lax
from jax.experimental import pallas as pl
from jax.experimental.pallas import tpu as pltpu
from jax.experimental.pallas import tpu_sc as plsc

_TILE_K = 4096


def _sc_gather_rows(table, idx):
    B = idx.shape[0]
    D = table.shape[1]
    info = plsc.get_sparse_core_info()
    nw = info.num_cores * info.num_subcores
    b_per_w = B // nw
    mesh = plsc.VectorSubcoreMesh(core_axis_name="c", subcore_axis_name="s")

    @functools.partial(
        pl.kernel, mesh=mesh,
        out_type=jax.ShapeDtypeStruct((B, D), jnp.float32),
        scratch_types=[
            pltpu.VMEM((b_per_w,), jnp.int32),
            pltpu.VMEM((b_per_w, D), jnp.float32),
            pltpu.SemaphoreType.DMA,
        ],
    )
    def gather_k(table_hbm, idx_hbm, out_hbm, idx_v, rows_v, sem):
        wid = lax.axis_index("s") * info.num_cores + lax.axis_index("c")
        base = wid * b_per_w
        pltpu.sync_copy(idx_hbm.at[pl.ds(base, b_per_w)], idx_v)
        pltpu.async_copy(table_hbm.at[idx_v], rows_v, sem).wait()
        pltpu.sync_copy(rows_v, out_hbm.at[pl.ds(base, b_per_w)])

    return gather_k(table, idx)


def _float_pred(x):
    xb = lax.bitcast_convert_type(x, jnp.int32)
    pb = jnp.where(xb == 0, jnp.int32(-2147483647),
                   jnp.where(xb > 0, xb - 1, xb + 1))
    return lax.bitcast_convert_type(pb, jnp.float32)


def _mrr_body(bands_ref, m1_ref, m2_ref, gath_ref, gt_ref, out_ref,
              sgt_ref, sgm_ref, cnt_ref, bufa_ref, bufb_ref, *, K, T, NT):
    k = pl.program_id(0)
    Q = m1_ref.shape[0]

    @pl.when(k == 0)
    def _groundtruth_scores():
        rows = lax.broadcasted_iota(jnp.int32, (Q, Q), 0)
        colq = lax.broadcasted_iota(jnp.int32, (Q, Q), 1)
        P = lax.dot_general(
            m1_ref[...], gath_ref[...],
            dimension_numbers=(((1,), (1,)), ((), ())),
            preferred_element_type=jnp.float32,
        )
        sg = jnp.sum(jnp.where(rows == colq, P, 0.0), axis=1, keepdims=True)
        sgt_ref[...] = sg
        sgm_ref[...] = _float_pred(sg)
        cnt_ref[...] = jnp.zeros_like(cnt_ref)
        bufb_ref[...] = jnp.full_like(bufb_ref, -jnp.inf)

    def count(dst_ref, src_ref, last):
        below = gt_ref[...] >= k * T
        thr = jnp.where(below, sgm_ref[...], sgt_ref[...])
        acc = jnp.zeros((Q, 1), jnp.float32)
        nc = 8
        C = T // nc
        for c in range(nc):
            sl = pl.ds(c * C, C)
            dst_ref[:, sl] = lax.dot_general(
                m1_ref[...], m2_ref[sl, :],
                dimension_numbers=(((1,), (1,)), ((), ())),
                preferred_element_type=jnp.float32,
            )
            sim = src_ref[:, sl]
            cmp = sim > thr
            if last:
                lane = c * C + lax.broadcasted_iota(jnp.int32, (Q, C), 1)
                cmp = jnp.logical_and(cmp, lane < K - (NT - 1) * T)
            acc = acc + jnp.sum(jnp.where(cmp, 1.0, 0.0), axis=1,
                                keepdims=True)
        cnt_ref[...] += acc
        t0 = jnp.maximum(k - 1, 0)
        lo = bands_ref[t0, 0]
        nch = jnp.where(k == 0, 0, bands_ref[t0, 1])

        def chunk(c, carry):
            r0 = pl.multiple_of(lo + c * 8, 8)
            simb = src_ref[pl.ds(r0, 8), :]
            sgb = sgt_ref[pl.ds(r0, 8), :]
            gth = gt_ref[pl.ds(r0, 8), :] - t0 * T
            gth = jnp.where(gth >= T, 0, gth)
            lane8 = lax.broadcasted_iota(jnp.int32, (8, T), 1)
            tie = jnp.logical_and(simb == sgb, lane8 < gth)
            cnt_ref[pl.ds(r0, 8), :] += jnp.sum(
                jnp.where(tie, 1.0, 0.0), axis=1, keepdims=True)
            return carry

        lax.fori_loop(0, nch, chunk, 0)

    def phase(dst_ref, src_ref):
        @pl.when(k < NT)
        def _steady():
            count(dst_ref, src_ref, last=False)

        @pl.when(k == NT)
        def _last():
            count(dst_ref, src_ref, last=True)

    @pl.when(k % 2 == 0)
    def _even():
        phase(bufa_ref, bufb_ref)

    @pl.when(k % 2 == 1)
    def _odd():
        phase(bufb_ref, bufa_ref)

    @pl.when(k == NT)
    def _finalize():
        ranks = cnt_ref[...] + 1.0
        out_ref[...] = jnp.mean(1.0 / ranks).reshape(1, 1)


def _mrr_g1(m1, m2, g):
    Q, D = m1.shape
    K = m2.shape[0]
    T = _TILE_K
    nt = pl.cdiv(K, T)

    order = jnp.argsort(g).astype(jnp.int32)
    g_sorted = jnp.take(g, order)

    gathered = _sc_gather_rows(m2, g_sorted)
    m1s = _sc_gather_rows(m1, order)

    tt = jnp.arange(nt, dtype=jnp.int32)
    lo = jnp.searchsorted(g_sorted, tt * T).astype(jnp.int32)
    hi = jnp.searchsorted(g_sorted, (tt + 1) * T).astype(jnp.int32)
    lo8 = lo // 8 * 8
    hi8 = jnp.minimum((hi + 7) // 8 * 8, Q)
    nch = jnp.maximum(hi8 - lo8, 0) // 8
    bands = jnp.stack([lo8, nch], axis=1)

    body = functools.partial(_mrr_body, K=K, T=T, NT=nt)
    grid_spec = pltpu.PrefetchScalarGridSpec(
        num_scalar_prefetch=1,
        grid=(nt + 1,),
        in_specs=[
            pl.BlockSpec((Q, D), lambda k, b: (0, 0)),
            pl.BlockSpec((T, D), lambda k, b: (jnp.minimum(k, nt - 1), 0)),
            pl.BlockSpec((Q, D), lambda k, b: (0, 0)),
            pl.BlockSpec((Q, 1), lambda k, b: (0, 0)),
        ],
        out_specs=pl.BlockSpec((1, 1), lambda k, b: (0, 0)),
        scratch_shapes=[
            pltpu.VMEM((Q, 1), jnp.float32),
            pltpu.VMEM((Q, 1), jnp.float32),
            pltpu.VMEM((Q, 1), jnp.float32),
            pltpu.VMEM((Q, T), jnp.float32),
            pltpu.VMEM((Q, T), jnp.float32),
        ],
    )
    out = pl.pallas_call(
        body,
        grid_spec=grid_spec,
        out_shape=jax.ShapeDtypeStruct((1, 1), jnp.float32),
    )(bands, m1s, m2, gathered, g_sorted.reshape(Q, 1))
    return out[0, 0]


def kernel(modality1_features, modality2_features, groundtruth_all_indices):
    gt = groundtruth_all_indices.astype(jnp.int32)
    Q, G = gt.shape
    if G != 1:
        raise NotImplementedError(
            "this problem's fixed shapes have one groundtruth per query")
    return _mrr_g1(modality1_features, modality2_features, gt[:, 0])


# speedup vs baseline: 1.4427x; 1.0699x over previous
import functools

import jax
import jax.numpy as jnp
from jax import lax
from jax.experimental import pallas as pl
from jax.experimental.pallas import tpu as pltpu
from jax.experimental.pallas import tpu_sc as plsc

_TILE_K = 4096


def _sc_gather_rows(table, idx):
    B = idx.shape[0]
    D = table.shape[1]
    info = plsc.get_sparse_core_info()
    nw = info.num_cores * info.num_subcores
    b_per_w = B // nw
    mesh = plsc.VectorSubcoreMesh(core_axis_name="c", subcore_axis_name="s")

    @functools.partial(
        pl.kernel, mesh=mesh,
        out_type=jax.ShapeDtypeStruct((B, D), jnp.float32),
        scratch_types=[
            pltpu.VMEM((b_per_w,), jnp.int32),
            pltpu.VMEM((b_per_w, D), jnp.float32),
            pltpu.SemaphoreType.DMA,
        ],
    )
    def gather_k(table_hbm, idx_hbm, out_hbm, idx_v, rows_v, sem):
        wid = lax.axis_index("s") * info.num_cores + lax.axis_index("c")
        base = wid * b_per_w
        pltpu.sync_copy(idx_hbm.at[pl.ds(base, b_per_w)], idx_v)
        pltpu.async_copy(table_hbm.at[idx_v], rows_v, sem).wait()
        pltpu.sync_copy(rows_v, out_hbm.at[pl.ds(base, b_per_w)])

    return gather_k(table, idx)


def _float_pred(x):
    xb = lax.bitcast_convert_type(x, jnp.int32)
    pb = jnp.where(xb == 0, jnp.int32(-2147483647),
                   jnp.where(xb > 0, xb - 1, xb + 1))
    return lax.bitcast_convert_type(pb, jnp.float32)


def _mrr_body(bands_ref, m1_ref, m2_ref, gath_ref, gt_ref, out_ref,
              sgt_ref, sgm_ref, cnt_ref, bufa_ref, bufb_ref, *, K, T, NT):
    k = pl.program_id(0)
    Q = m1_ref.shape[0]

    @pl.when(k == 0)
    def _groundtruth_scores():
        rows = lax.broadcasted_iota(jnp.int32, (Q, Q), 0)
        colq = lax.broadcasted_iota(jnp.int32, (Q, Q), 1)
        P = lax.dot_general(
            m1_ref[...], gath_ref[...],
            dimension_numbers=(((1,), (1,)), ((), ())),
            preferred_element_type=jnp.float32,
        )
        sg = jnp.sum(jnp.where(rows == colq, P, 0.0), axis=1, keepdims=True)
        sgt_ref[...] = sg
        sgm_ref[...] = _float_pred(sg)
        cnt_ref[...] = jnp.zeros_like(cnt_ref)
        bufb_ref[...] = jnp.full_like(bufb_ref, -jnp.inf)

    def count(dst_ref, src_ref, last):
        below = gt_ref[...] >= k * T
        thr = jnp.where(below, sgm_ref[...], sgt_ref[...])
        acc = jnp.zeros((Q, 1), jnp.float32)
        nc = 16
        C = T // nc
        for c in range(nc):
            sl = pl.ds(c * C, C)
            dst_ref[:, sl] = lax.dot_general(
                m1_ref[...], m2_ref[sl, :],
                dimension_numbers=(((1,), (1,)), ((), ())),
                preferred_element_type=jnp.float32,
            )
            sim = src_ref[:, sl]
            cmp = sim > thr
            if last:
                lane = c * C + lax.broadcasted_iota(jnp.int32, (Q, C), 1)
                cmp = jnp.logical_and(cmp, lane < K - (NT - 1) * T)
            acc = acc + jnp.sum(jnp.where(cmp, 1.0, 0.0), axis=1,
                                keepdims=True)
        cnt_ref[...] += acc
        t0 = jnp.maximum(k - 1, 0)
        lo = bands_ref[t0, 0]
        nch = jnp.where(k == 0, 0, bands_ref[t0, 1])

        def chunk(c, carry):
            r0 = pl.multiple_of(lo + c * 8, 8)
            simb = src_ref[pl.ds(r0, 8), :]
            sgb = sgt_ref[pl.ds(r0, 8), :]
            gth = gt_ref[pl.ds(r0, 8), :] - t0 * T
            gth = jnp.where(gth >= T, 0, gth)
            lane8 = lax.broadcasted_iota(jnp.int32, (8, T), 1)
            tie = jnp.logical_and(simb == sgb, lane8 < gth)
            cnt_ref[pl.ds(r0, 8), :] += jnp.sum(
                jnp.where(tie, 1.0, 0.0), axis=1, keepdims=True)
            return carry

        lax.fori_loop(0, nch, chunk, 0)

    def phase(dst_ref, src_ref):
        @pl.when(k < NT)
        def _steady():
            count(dst_ref, src_ref, last=False)

        @pl.when(k == NT)
        def _last():
            count(dst_ref, src_ref, last=True)

    @pl.when(k % 2 == 0)
    def _even():
        phase(bufa_ref, bufb_ref)

    @pl.when(k % 2 == 1)
    def _odd():
        phase(bufb_ref, bufa_ref)

    @pl.when(k == NT)
    def _finalize():
        ranks = cnt_ref[...] + 1.0
        out_ref[...] = jnp.mean(1.0 / ranks).reshape(1, 1)


def _mrr_g1(m1, m2, g):
    Q, D = m1.shape
    K = m2.shape[0]
    T = _TILE_K
    nt = pl.cdiv(K, T)

    order = jnp.argsort(g).astype(jnp.int32)
    g_sorted = jnp.take(g, order)

    gathered = _sc_gather_rows(m2, g_sorted)
    m1s = _sc_gather_rows(m1, order)

    tt = jnp.arange(nt, dtype=jnp.int32)
    lo = jnp.searchsorted(g_sorted, tt * T).astype(jnp.int32)
    hi = jnp.searchsorted(g_sorted, (tt + 1) * T).astype(jnp.int32)
    lo8 = lo // 8 * 8
    hi8 = jnp.minimum((hi + 7) // 8 * 8, Q)
    nch = jnp.maximum(hi8 - lo8, 0) // 8
    bands = jnp.stack([lo8, nch], axis=1)

    body = functools.partial(_mrr_body, K=K, T=T, NT=nt)
    grid_spec = pltpu.PrefetchScalarGridSpec(
        num_scalar_prefetch=1,
        grid=(nt + 1,),
        in_specs=[
            pl.BlockSpec((Q, D), lambda k, b: (0, 0)),
            pl.BlockSpec((T, D), lambda k, b: (jnp.minimum(k, nt - 1), 0)),
            pl.BlockSpec((Q, D), lambda k, b: (0, 0)),
            pl.BlockSpec((Q, 1), lambda k, b: (0, 0)),
        ],
        out_specs=pl.BlockSpec((1, 1), lambda k, b: (0, 0)),
        scratch_shapes=[
            pltpu.VMEM((Q, 1), jnp.float32),
            pltpu.VMEM((Q, 1), jnp.float32),
            pltpu.VMEM((Q, 1), jnp.float32),
            pltpu.VMEM((Q, T), jnp.float32),
            pltpu.VMEM((Q, T), jnp.float32),
        ],
    )
    out = pl.pallas_call(
        body,
        grid_spec=grid_spec,
        out_shape=jax.ShapeDtypeStruct((1, 1), jnp.float32),
    )(bands, m1s, m2, gathered, g_sorted.reshape(Q, 1))
    return out[0, 0]


def kernel(modality1_features, modality2_features, groundtruth_all_indices):
    gt = groundtruth_all_indices.astype(jnp.int32)
    Q, G = gt.shape
    if G != 1:
        raise NotImplementedError(
            "this problem's fixed shapes have one groundtruth per query")
    return _mrr_g1(modality1_features, modality2_features, gt[:, 0])
